# baseline (speedup 1.0000x reference)
import jax
import jax.numpy as jnp
from jax import lax
from jax.experimental import pallas as pl
from jax.experimental.pallas import tpu as pltpu
from jax.experimental.pallas import tpu_sc as plsc

B = 4096
T = 200
D = 32
VOCAB = 1000000
POS_SIZE = 200
NUM_SAMPLED = 5

NC = 2
NS = 16
NW = NC * NS
BT = B * T
SPAN = BT // NW
BW = B // NW


def _sc_gather_body(item_table, items_flat, pos_flat, keys_flat, tgt_flat,
                    pos_tab_flat, wp_in, samp_in,
                    rows_out, ps_out, ids_out, truew_out, tgtw_out, res_out,
                    sampw_out,
                    idxvm, posvm, psbuf, idsbuf, ptvm, pwvm, wpvm,
                    tidx, rowbuf, rowbufT, klvm, tivm, resvm, twbuf,
                    sampvm, swbuf, gsem, wsem):
    wid = lax.axis_index("s") * NC + lax.axis_index("c")
    base = wid * SPAN
    bb = wid * BW

    pltpu.sync_copy(items_flat.at[pl.ds(base, SPAN)], idxvm)
    pltpu.sync_copy(pos_flat.at[pl.ds(base, SPAN)], posvm)
    pltpu.sync_copy(pos_tab_flat, ptvm.at[pl.ds(0, POS_SIZE * D)])
    pltpu.sync_copy(wp_in, wpvm)

    wv = (wpvm[pl.ds(0, 16)], wpvm[pl.ds(16, 16)])

    def _pw_chunk(c, _):
        pvec = lax.iota(jnp.int32, 16) + c * 16
        acc = jnp.zeros((16,), jnp.float32)
        for ch in range(D):
            col = plsc.load_gather(ptvm, [pvec * D + ch])
            acc = acc + col * wv[ch // 16][ch % 16]
        pwvm[pl.ds(c * 16, 16)] = acc
        return 0
    lax.fori_loop(0, 13, _pw_chunk, 0, unroll=False)

    lanes16 = lax.iota(jnp.int32, 16)

    def _transpose128(src, dst):
        def _diag(d, _):
            cvec = lax.bitwise_and(lanes16 + d, D - 1)
            for k in range(BW // 16):
                rvec = lanes16 + k * 16
                v = plsc.load_gather(src, [rvec, cvec])
                plsc.store_scatter(dst, [cvec, rvec], v)
            return 0
        lax.fori_loop(0, D, _diag, 0, unroll=False)

    def _chunk(t, _):
        for k in range(BW // 16):
            loc = (lax.iota(jnp.int32, 16) + k * 16) * T + t
            iv = plsc.load_gather(idxvm, [loc])
            tidx[pl.ds(k * 16, 16)] = iv
            idsbuf[t, pl.ds(k * 16, 16)] = iv
            pv = plsc.load_gather(posvm, [loc])
            psbuf[t, pl.ds(k * 16, 16)] = plsc.load_gather(pwvm, [pv])
        cp = pltpu.async_copy(item_table.at[tidx], rowbuf, gsem)
        cp.wait()
        _transpose128(rowbuf, rowbufT)
        pltpu.sync_copy(rowbufT, rows_out.at[wid, t])
        return 0
    lax.fori_loop(0, T, _chunk, 0, unroll=False)
    pltpu.sync_copy(psbuf, ps_out.at[wid])
    pltpu.sync_copy(idsbuf, ids_out.at[wid])

    pltpu.sync_copy(keys_flat.at[pl.ds(bb, BW)], klvm)
    pltpu.sync_copy(tgt_flat.at[pl.ds(bb, BW)], tivm)
    for k in range(BW // 16):
        lane = lax.iota(jnp.int32, 16) + k * 16
        lenv = klvm[pl.ds(k * 16, 16)]
        off = lane * T + lenv - 1
        resvm[pl.ds(k * 16, 16)] = plsc.load_gather(idxvm, [off])
    pltpu.sync_copy(resvm, res_out.at[wid, 0])
    pltpu.async_copy(item_table.at[resvm], twbuf, wsem).wait()
    _transpose128(twbuf, rowbufT)
    pltpu.sync_copy(rowbufT, truew_out.at[wid])
    pltpu.async_copy(item_table.at[tivm], twbuf, wsem).wait()
    _transpose128(twbuf, rowbufT)
    pltpu.sync_copy(rowbufT, tgtw_out.at[wid])

    @pl.when(wid == 0)
    def _():
        pltpu.sync_copy(samp_in, sampvm)
        pltpu.async_copy(item_table.at[sampvm], swbuf, wsem).wait()
        pltpu.sync_copy(swbuf, sampw_out.at[:, pl.ds(0, D)])


def _sc_gather(item_table, items_id, position_id, keys_length, target_id,
               pos_table, wp, samp8):
    mesh = plsc.VectorSubcoreMesh(core_axis_name="c", subcore_axis_name="s",
                                  num_cores=NC, num_subcores=NS)
    out_type = (
        jax.ShapeDtypeStruct((NW, T, D, 128), jnp.float32),
        jax.ShapeDtypeStruct((NW, T, 128), jnp.float32),
        jax.ShapeDtypeStruct((NW, T, 128), jnp.int32),
        jax.ShapeDtypeStruct((NW, D, 128), jnp.float32),
        jax.ShapeDtypeStruct((NW, D, 128), jnp.float32),
        jax.ShapeDtypeStruct((NW, 1, 128), jnp.int32),
        jax.ShapeDtypeStruct((8, 128), jnp.float32),
    )
    scratch = [
        pltpu.VMEM((SPAN,), jnp.int32),
        pltpu.VMEM((SPAN,), jnp.int32),
        pltpu.VMEM((T, BW), jnp.float32),
        pltpu.VMEM((T, BW), jnp.int32),
        pltpu.VMEM((POS_SIZE * D + 256,), jnp.float32),
        pltpu.VMEM((208,), jnp.float32),
        pltpu.VMEM((D,), jnp.float32),
        pltpu.VMEM((BW,), jnp.int32),
        pltpu.VMEM((BW, D), jnp.float32),
        pltpu.VMEM((D, BW), jnp.float32),
        pltpu.VMEM((BW,), jnp.int32),
        pltpu.VMEM((BW,), jnp.int32),
        pltpu.VMEM((BW,), jnp.int32),
        pltpu.VMEM((BW, D), jnp.float32),
        pltpu.VMEM((8,), jnp.int32),
        pltpu.VMEM((8, D), jnp.float32),
        pltpu.SemaphoreType.DMA,
        pltpu.SemaphoreType.DMA,
    ]
    fn = pl.kernel(_sc_gather_body, out_type=out_type, mesh=mesh,
                   scratch_types=scratch,
                   compiler_params=pltpu.CompilerParams(
                       needs_layout_passes=False,
                       use_tc_tiling_on_sc=False))
    return fn(item_table, items_id, position_id, keys_length, target_id,
              pos_table.reshape(POS_SIZE * D), wp, samp8)


def _tc_body(rows_ref, ps_ref, ids_ref, keys_ref, truew_ref, tgtw_ref,
             res_ref, sampw_ref, sampid_ref, wdT_ref, alphaT_ref, attb_ref,
             out_ref, loss_ref):
    i = pl.program_id(0)

    rows = rows_ref[...][0]
    s = jnp.sum(rows * wdT_ref[...][None], axis=1)
    s = s + ps_ref[...][0] + attb_ref[0, 0]
    sc = jnp.tanh(s)

    tt = lax.broadcasted_iota(jnp.int32, s.shape, 0)
    valid = tt < keys_ref[...][0]
    e = jnp.where(valid, jnp.exp(sc), 0.0)
    z = jnp.sum(e, axis=0, keepdims=True)

    pad = (ids_ref[...][0] != 0) & valid & (tt < T - 1)
    e2 = jnp.where(pad, e, 0.0)

    A = jnp.sum(e[:, None, :] * rows, axis=0)
    A2 = jnp.sum(e2[:, None, :] * rows, axis=0)

    pool = A / z
    pool = jnp.maximum(pool, 0.0) + alphaT_ref[...] * jnp.minimum(pool, 0.0)
    out_ref[...] = jnp.sum(pool * tgtw_ref[...][0], axis=0,
                           keepdims=True)[None]

    ut1 = A2 / z

    logv = jnp.log(float(VOCAB) + 1.0)

    def logq(f):
        return jnp.log((jnp.log(f + 2.0) - jnp.log(f + 1.0)) / logv)

    rf = res_ref[...][0].astype(jnp.float32)
    tl = jnp.sum(ut1 * truew_ref[...][0], axis=0, keepdims=True) - logq(rf)

    sq = logq(sampid_ref[...].astype(jnp.float32))
    m = tl
    sls = []
    for k in range(NUM_SAMPLED):
        swk = sampw_ref[k:k + 1, 0:D]
        slk = lax.dot_general(swk, ut1, (((1,), (0,)), ((), ())),
                              preferred_element_type=jnp.float32)
        slk = slk - sq[0:1, k:k + 1]
        sls.append(slk)
        m = jnp.maximum(m, slk)
    ssum = jnp.exp(tl - m)
    for slk in sls:
        ssum = ssum + jnp.exp(slk - m)
    lse = m + jnp.log(ssum)
    part = jnp.sum(lse - tl, axis=1, keepdims=True) * (1.0 / B)

    @pl.when(i == 0)
    def _():
        loss_ref[...] = jnp.zeros((1, 1), jnp.float32)
    loss_ref[...] += part


def _tc_compute(rows, ps, ids, keys3, truew, tgtw, res3,
                sampw, sampid, wdT, alphaT, attb):
    grid = (NW,)
    bspec = pl.BlockSpec
    out, loss = pl.pallas_call(
        _tc_body,
        grid=grid,
        in_specs=[
            bspec((1, T, D, 128), lambda i: (i, 0, 0, 0)),
            bspec((1, T, 128), lambda i: (i, 0, 0)),
            bspec((1, T, 128), lambda i: (i, 0, 0)),
            bspec((1, 1, 128), lambda i: (i, 0, 0)),
            bspec((1, D, 128), lambda i: (i, 0, 0)),
            bspec((1, D, 128), lambda i: (i, 0, 0)),
            bspec((1, 1, 128), lambda i: (i, 0, 0)),
            bspec((8, 128), lambda i: (0, 0)),
            bspec((1, 8), lambda i: (0, 0)),
            bspec((D, 1), lambda i: (0, 0)),
            bspec((D, 1), lambda i: (0, 0)),
            bspec((1, 1), lambda i: (0, 0)),
        ],
        out_specs=[
            bspec((1, 1, 128), lambda i: (i, 0, 0)),
            bspec((1, 1), lambda i: (0, 0)),
        ],
        out_shape=[
            jax.ShapeDtypeStruct((NW, 1, 128), jnp.float32),
            jax.ShapeDtypeStruct((1, 1), jnp.float32),
        ],
    )(rows, ps, ids, keys3, truew, tgtw, res3, sampw, sampid,
      wdT, alphaT, attb)
    return out, loss


@jax.jit
def kernel(items_id, position_id, target_id, keys_length, item_table,
           pos_table, att_W, att_b, prelu_alpha, zero_bias, sampled_ids):
    items_flat = items_id.reshape(BT)
    pos_flat = position_id.reshape(BT)
    item_table = item_table + lax.optimization_barrier(jnp.float32(0.0))
    wdT = att_W[:D, :]
    wp = att_W[D:, 0]
    samp8 = jnp.pad(sampled_ids, (0, 8 - NUM_SAMPLED))
    rows, ps, idsT, truew, tgtw, res3, sampw = _sc_gather(
        item_table, items_flat, pos_flat, keys_length[:, 0], target_id[:, 0],
        pos_table, wp, samp8)
    alphaT = prelu_alpha.reshape(D, 1)
    out, loss = _tc_compute(
        rows, ps, idsT, keys_length.reshape(NW, 1, 128), truew, tgtw,
        res3, sampw, samp8.reshape(1, 8),
        wdT, alphaT, att_b.reshape(1, 1))
    return out.reshape(B, 1), loss[0, 0]


# speedup vs baseline: 1.5550x; 1.5550x over previous
import jax
import jax.numpy as jnp
from jax import lax
from jax.experimental import pallas as pl
from jax.experimental.pallas import tpu as pltpu
from jax.experimental.pallas import tpu_sc as plsc

B = 4096
T = 200
D = 32
VOCAB = 1000000
POS_SIZE = 200
NUM_SAMPLED = 5

NC = 2
NS = 16
NW = NC * NS
BT = B * T
SPAN = BT // NW
BW = B // NW


def _sc_gather_body(item_table, items_flat, pos_flat, keys_flat, tgt_flat,
                    pos_tab_flat, wp_in, samp_in,
                    rows_out, ps_out, ids_out, truew_out, tgtw_out, res_out,
                    sampw_out,
                    idxvm, posvm, psbuf, idsbuf, ptvm, pwvm, wpvm,
                    tidx, rowbuf, tidxb, rowbufb, rowbufT, klvm, tivm, resvm,
                    twbuf, sampvm, swbuf, gsem, wsem, gsemb):
    wid = lax.axis_index("s") * NC + lax.axis_index("c")
    base = wid * SPAN
    bb = wid * BW

    pltpu.sync_copy(items_flat.at[pl.ds(base, SPAN)], idxvm)
    pltpu.sync_copy(pos_flat.at[pl.ds(base, SPAN)], posvm)
    pltpu.sync_copy(pos_tab_flat, ptvm.at[pl.ds(0, POS_SIZE * D)])
    pltpu.sync_copy(wp_in, wpvm)

    wv = (wpvm[pl.ds(0, 16)], wpvm[pl.ds(16, 16)])

    def _pw_chunk(c, _):
        pvec = lax.iota(jnp.int32, 16) + c * 16
        acc = jnp.zeros((16,), jnp.float32)
        for ch in range(D):
            col = plsc.load_gather(ptvm, [pvec * D + ch])
            acc = acc + col * wv[ch // 16][ch % 16]
        pwvm[pl.ds(c * 16, 16)] = acc
        return 0
    lax.fori_loop(0, 13, _pw_chunk, 0, unroll=False)

    lanes16 = lax.iota(jnp.int32, 16)

    def _transpose128(src, dst):
        def _diag(d, _):
            cvec = lax.bitwise_and(lanes16 + d, D - 1)
            for k in range(BW // 16):
                rvec = lanes16 + k * 16
                v = plsc.load_gather(src, [rvec, cvec])
                plsc.store_scatter(dst, [cvec, rvec], v)
            return 0
        lax.fori_loop(0, D, _diag, 0, unroll=False)

    def _start(ti, rb, sem, t):
        for k in range(BW // 16):
            loc = (lanes16 + k * 16) * T + t
            ti[pl.ds(k * 16, 16)] = plsc.load_gather(idxvm, [loc])
        pltpu.async_copy(item_table.at[ti], rb, sem)

    def _psids(ti, t):
        for k in range(BW // 16):
            loc = (lanes16 + k * 16) * T + t
            idsbuf[t, pl.ds(k * 16, 16)] = ti[pl.ds(k * 16, 16)]
            pv = plsc.load_gather(posvm, [loc])
            psbuf[t, pl.ds(k * 16, 16)] = plsc.load_gather(pwvm, [pv])

    def _finish(ti, rb, sem, t):
        pltpu.make_async_copy(item_table.at[ti], rb, sem).wait()
        _transpose128(rb, rowbufT)
        pltpu.sync_copy(rowbufT, rows_out.at[wid, t])

    _start(tidx, rowbuf, gsem, 0)

    def _pair(i, _):
        ta = 2 * i
        _start(tidxb, rowbufb, gsemb, ta + 1)
        _psids(tidx, ta)
        _finish(tidx, rowbuf, gsem, ta)

        @pl.when(i < T // 2 - 1)
        def _():
            _start(tidx, rowbuf, gsem, ta + 2)
        _psids(tidxb, ta + 1)
        _finish(tidxb, rowbufb, gsemb, ta + 1)
        return 0
    lax.fori_loop(0, T // 2, _pair, 0, unroll=False)
    pltpu.sync_copy(psbuf, ps_out.at[wid])
    pltpu.sync_copy(idsbuf, ids_out.at[wid])

    pltpu.sync_copy(keys_flat.at[pl.ds(bb, BW)], klvm)
    pltpu.sync_copy(tgt_flat.at[pl.ds(bb, BW)], tivm)
    for k in range(BW // 16):
        lane = lax.iota(jnp.int32, 16) + k * 16
        lenv = klvm[pl.ds(k * 16, 16)]
        off = lane * T + lenv - 1
        resvm[pl.ds(k * 16, 16)] = plsc.load_gather(idxvm, [off])
    pltpu.sync_copy(resvm, res_out.at[wid, 0])
    pltpu.async_copy(item_table.at[resvm], twbuf, wsem).wait()
    _transpose128(twbuf, rowbufT)
    pltpu.sync_copy(rowbufT, truew_out.at[wid])
    pltpu.async_copy(item_table.at[tivm], twbuf, wsem).wait()
    _transpose128(twbuf, rowbufT)
    pltpu.sync_copy(rowbufT, tgtw_out.at[wid])

    @pl.when(wid == 0)
    def _():
        pltpu.sync_copy(samp_in, sampvm)
        pltpu.async_copy(item_table.at[sampvm], swbuf, wsem).wait()
        pltpu.sync_copy(swbuf, sampw_out.at[:, pl.ds(0, D)])


def _sc_gather(item_table, items_id, position_id, keys_length, target_id,
               pos_table, wp, samp8):
    mesh = plsc.VectorSubcoreMesh(core_axis_name="c", subcore_axis_name="s",
                                  num_cores=NC, num_subcores=NS)
    out_type = (
        jax.ShapeDtypeStruct((NW, T, D, 128), jnp.float32),
        jax.ShapeDtypeStruct((NW, T, 128), jnp.float32),
        jax.ShapeDtypeStruct((NW, T, 128), jnp.int32),
        jax.ShapeDtypeStruct((NW, D, 128), jnp.float32),
        jax.ShapeDtypeStruct((NW, D, 128), jnp.float32),
        jax.ShapeDtypeStruct((NW, 1, 128), jnp.int32),
        jax.ShapeDtypeStruct((8, 128), jnp.float32),
    )
    scratch = [
        pltpu.VMEM((SPAN,), jnp.int32),
        pltpu.VMEM((SPAN,), jnp.int32),
        pltpu.VMEM((T, BW), jnp.float32),
        pltpu.VMEM((T, BW), jnp.int32),
        pltpu.VMEM((POS_SIZE * D + 256,), jnp.float32),
        pltpu.VMEM((208,), jnp.float32),
        pltpu.VMEM((D,), jnp.float32),
        pltpu.VMEM((BW,), jnp.int32),
        pltpu.VMEM((BW, D), jnp.float32),
        pltpu.VMEM((BW,), jnp.int32),
        pltpu.VMEM((BW, D), jnp.float32),
        pltpu.VMEM((D, BW), jnp.float32),
        pltpu.VMEM((BW,), jnp.int32),
        pltpu.VMEM((BW,), jnp.int32),
        pltpu.VMEM((BW,), jnp.int32),
        pltpu.VMEM((BW, D), jnp.float32),
        pltpu.VMEM((8,), jnp.int32),
        pltpu.VMEM((8, D), jnp.float32),
        pltpu.SemaphoreType.DMA,
        pltpu.SemaphoreType.DMA,
        pltpu.SemaphoreType.DMA,
    ]
    fn = pl.kernel(_sc_gather_body, out_type=out_type, mesh=mesh,
                   scratch_types=scratch,
                   compiler_params=pltpu.CompilerParams(
                       needs_layout_passes=False,
                       use_tc_tiling_on_sc=False))
    return fn(item_table, items_id, position_id, keys_length, target_id,
              pos_table.reshape(POS_SIZE * D), wp, samp8)


def _tc_body(rows_ref, ps_ref, ids_ref, keys_ref, truew_ref, tgtw_ref,
             res_ref, sampw_ref, sampid_ref, wdT_ref, alphaT_ref, attb_ref,
             out_ref, loss_ref):
    i = pl.program_id(0)

    rows = rows_ref[...][0]
    s = jnp.sum(rows * wdT_ref[...][None], axis=1)
    s = s + ps_ref[...][0] + attb_ref[0, 0]
    sc = jnp.tanh(s)

    tt = lax.broadcasted_iota(jnp.int32, s.shape, 0)
    valid = tt < keys_ref[...][0]
    e = jnp.where(valid, jnp.exp(sc), 0.0)
    z = jnp.sum(e, axis=0, keepdims=True)

    pad = (ids_ref[...][0] != 0) & valid & (tt < T - 1)
    e2 = jnp.where(pad, e, 0.0)

    A = jnp.sum(e[:, None, :] * rows, axis=0)
    A2 = jnp.sum(e2[:, None, :] * rows, axis=0)

    pool = A / z
    pool = jnp.maximum(pool, 0.0) + alphaT_ref[...] * jnp.minimum(pool, 0.0)
    out_ref[...] = jnp.sum(pool * tgtw_ref[...][0], axis=0,
                           keepdims=True)[None]

    ut1 = A2 / z

    logv = jnp.log(float(VOCAB) + 1.0)

    def logq(f):
        return jnp.log((jnp.log(f + 2.0) - jnp.log(f + 1.0)) / logv)

    rf = res_ref[...][0].astype(jnp.float32)
    tl = jnp.sum(ut1 * truew_ref[...][0], axis=0, keepdims=True) - logq(rf)

    sq = logq(sampid_ref[...].astype(jnp.float32))
    m = tl
    sls = []
    for k in range(NUM_SAMPLED):
        swk = sampw_ref[k:k + 1, 0:D]
        slk = lax.dot_general(swk, ut1, (((1,), (0,)), ((), ())),
                              preferred_element_type=jnp.float32)
        slk = slk - sq[0:1, k:k + 1]
        sls.append(slk)
        m = jnp.maximum(m, slk)
    ssum = jnp.exp(tl - m)
    for slk in sls:
        ssum = ssum + jnp.exp(slk - m)
    lse = m + jnp.log(ssum)
    part = jnp.sum(lse - tl, axis=1, keepdims=True) * (1.0 / B)

    @pl.when(i == 0)
    def _():
        loss_ref[...] = jnp.zeros((1, 1), jnp.float32)
    loss_ref[...] += part


def _tc_compute(rows, ps, ids, keys3, truew, tgtw, res3,
                sampw, sampid, wdT, alphaT, attb):
    grid = (NW,)
    bspec = pl.BlockSpec
    out, loss = pl.pallas_call(
        _tc_body,
        grid=grid,
        in_specs=[
            bspec((1, T, D, 128), lambda i: (i, 0, 0, 0)),
            bspec((1, T, 128), lambda i: (i, 0, 0)),
            bspec((1, T, 128), lambda i: (i, 0, 0)),
            bspec((1, 1, 128), lambda i: (i, 0, 0)),
            bspec((1, D, 128), lambda i: (i, 0, 0)),
            bspec((1, D, 128), lambda i: (i, 0, 0)),
            bspec((1, 1, 128), lambda i: (i, 0, 0)),
            bspec((8, 128), lambda i: (0, 0)),
            bspec((1, 8), lambda i: (0, 0)),
            bspec((D, 1), lambda i: (0, 0)),
            bspec((D, 1), lambda i: (0, 0)),
            bspec((1, 1), lambda i: (0, 0)),
        ],
        out_specs=[
            bspec((1, 1, 128), lambda i: (i, 0, 0)),
            bspec((1, 1), lambda i: (0, 0)),
        ],
        out_shape=[
            jax.ShapeDtypeStruct((NW, 1, 128), jnp.float32),
            jax.ShapeDtypeStruct((1, 1), jnp.float32),
        ],
    )(rows, ps, ids, keys3, truew, tgtw, res3, sampw, sampid,
      wdT, alphaT, attb)
    return out, loss


@jax.jit
def kernel(items_id, position_id, target_id, keys_length, item_table,
           pos_table, att_W, att_b, prelu_alpha, zero_bias, sampled_ids):
    items_flat = items_id.reshape(BT)
    pos_flat = position_id.reshape(BT)
    wdT = att_W[:D, :]
    wp = att_W[D:, 0]
    samp8 = jnp.pad(sampled_ids, (0, 8 - NUM_SAMPLED))
    rows, ps, idsT, truew, tgtw, res3, sampw = _sc_gather(
        item_table, items_flat, pos_flat, keys_length[:, 0], target_id[:, 0],
        pos_table, wp, samp8)
    alphaT = prelu_alpha.reshape(D, 1)
    out, loss = _tc_compute(
        rows, ps, idsT, keys_length.reshape(NW, 1, 128), truew, tgtw,
        res3, sampw, samp8.reshape(1, 8),
        wdT, alphaT, att_b.reshape(1, 1))
    return out.reshape(B, 1), loss[0, 0]
